# baseline (speedup 1.0000x reference)
import functools

import jax
import jax.numpy as jnp
from jax import lax
from jax.experimental import pallas as pl
from jax.experimental.pallas import tpu as pltpu
from jax.experimental.pallas import tpu_sc as plsc

_TOPK = 0.1
_KH, _KW = 5, 5
_INT_MAX = 2147483647
_SIGN = -2147483648
_NC, _NS = 2, 16
_NW = _NC * _NS
_CH = 48


def _sortable_key(f):
    b = jax.lax.bitcast_convert_type(f, jnp.int32)
    return jnp.where(b < 0, b ^ 0x7FFFFFFF, b)


def _mean_body(x_ref, t_ref):
    t_ref[0] = jnp.mean(x_ref[0], axis=0)


def _sc_select_body(t_hbm, kth_hbm, rowbuf, hfine, hcoarse, outv, *, n, h, w,
                    hout, wout, kkeep, nrounds):
    nprob = n * _KH * _KW
    wid = lax.axis_index("s") * _NC + lax.axis_index("c")
    lane = lax.broadcasted_iota(jnp.int32, (16,), 0)
    zeros16 = jnp.zeros((16,), jnp.int32)
    nvec = w // 16

    def walk(hist_ref, base, nbins, krank):
        def step(t, carry):
            tot, bstar, kin = carry
            bin_ = base + (nbins - 1 - t)
            vec = plsc.load_gather(hist_ref, [bin_ * 16 + lane])
            s = jnp.sum(vec)
            found = jnp.logical_and(tot < krank, tot + s >= krank)
            bstar = jnp.where(found, bin_, bstar)
            kin = jnp.where(found, krank - tot, kin)
            return tot + s, bstar, kin
        _, bstar, kin = lax.fori_loop(
            0, nbins, step, (jnp.int32(0), jnp.int32(-1), jnp.int32(0)))
        return bstar, kin

    def scan_pass(b, wi, wj, rank, shift, pfx_shift, pfx):
        def clr_f(i, c):
            plsc.store_scatter(hfine, [i * 16 + lane], zeros16)
            return c
        lax.fori_loop(0, 4096, clr_f, 0)

        def clr_c(i, c):
            plsc.store_scatter(hcoarse, [i * 16 + lane], zeros16)
            return c
        lax.fori_loop(0, 256, clr_c, 0)

        def chunk_body(ck, c):
            pltpu.sync_copy(t_hbm.at[pl.ds(b * h + ck * _CH, _CH)], rowbuf)

            def row_body(r, c2):
                grow = ck * _CH + r
                row_ok = jnp.logical_and(grow >= wi, grow < wi + hout)
                rvec = jnp.full((16,), r, jnp.int32)

                def vec_body(v, c3):
                    col = v * 16 + lane
                    xv = plsc.load_gather(rowbuf, [rvec, col])
                    bits = lax.bitcast_convert_type(xv, jnp.int32)
                    ukey = jnp.where(bits < 0, ~bits, bits ^ _SIGN)
                    ok = jnp.logical_and(col >= wj, col < wj + wout)
                    ok = jnp.logical_and(ok, row_ok)
                    if shift == 20:
                        sl = lax.shift_right_logical(ukey, 20)
                    elif shift == 8:
                        sl = jnp.bitwise_and(
                            lax.shift_right_logical(ukey, 8), 0xFFF)
                    else:
                        sl = jnp.left_shift(jnp.bitwise_and(ukey, 0xFF), 4)
                    if pfx is not None:
                        ok = jnp.logical_and(
                            ok, lax.shift_right_logical(ukey, pfx_shift)
                            == pfx)
                    val = ok.astype(jnp.int32)
                    cidx = lax.shift_right_logical(sl, 4)
                    plsc.addupdate_scatter(hcoarse, [cidx * 16 + lane], val)
                    plsc.addupdate_scatter(hfine, [sl * 16 + lane], val)
                    return c3
                lax.fori_loop(0, nvec, vec_body, 0)
                return c2
            lax.fori_loop(0, _CH, row_body, 0)
            return c
        lax.fori_loop(0, h // _CH, chunk_body, 0)

        cstar, kin = walk(hcoarse, jnp.int32(0), 256, rank)
        fstar, kin2 = walk(hfine, cstar * 16, 16, kin)
        return fstar, kin2

    def round_body(rnd, carry):
        p = rnd * _NW + wid

        @pl.when(p < nprob)
        def _():
            b = p // (_KH * _KW)
            wwin = p % (_KH * _KW)
            wi = wwin // _KW
            wj = wwin % _KW
            f1, k2 = scan_pass(b, wi, wj, jnp.int32(kkeep), 20, 0, None)
            f2, k3 = scan_pass(b, wi, wj, k2, 8, 20, f1)
            pfx2 = jnp.bitwise_or(jnp.left_shift(f1, 12), f2)
            f3, _ = scan_pass(b, wi, wj, k3, 0, 8, pfx2)
            ukey = jnp.bitwise_or(jnp.left_shift(pfx2, 8),
                                  lax.shift_right_logical(f3, 4))
            skey = ukey ^ _SIGN
            outv[...] = jnp.full((16,), skey, jnp.int32)
            pltpu.sync_copy(outv, kth_hbm.at[pl.ds(p * 16, 16)])
        return carry

    lax.fori_loop(0, nrounds, round_body, 0)


def _apply_body(x_ref, t_ref, k_ref, tau_ref, o_ref, *, hout, wout, ht):
    i0 = pl.program_id(1)
    T = t_ref[0]
    H, W = T.shape
    key = _sortable_key(T)

    p = i0 * ht + jax.lax.broadcasted_iota(jnp.int32, (H, 1), 0)
    q = jax.lax.broadcasted_iota(jnp.int32, (1, W), 1)
    big = jnp.int32(_INT_MAX)
    thr = None
    for j in range(_KW):
        cm = jnp.logical_and(q >= j, q < j + wout)
        rmin = None
        for i in range(_KH):
            rm = jnp.logical_and(p >= i, p < i + hout)
            v = jnp.where(rm, k_ref[0, 0, i * _KW + j], big)
            rmin = v if rmin is None else jnp.minimum(rmin, v)
        v = jnp.where(cm, rmin, big)
        thr = v if thr is None else jnp.minimum(thr, v)

    mask = (key >= thr).astype(jnp.float32)
    tau = tau_ref[0, 0]
    wmap = mask * tau + (1.0 - tau)
    o_ref[0] = x_ref[0] * wmap[None]


@jax.jit
def kernel(x, tau):
    n, c, h, w = x.shape
    hout, wout = h - _KH + 1, w - _KW + 1
    kkeep = max(int(_TOPK * (hout * wout)), 1)
    nprob = n * _KH * _KW
    nrounds = (nprob + _NW - 1) // _NW

    ht = 32 if h % 32 == 0 else h
    nh = h // ht

    tmean = pl.pallas_call(
        _mean_body,
        grid=(n, nh),
        in_specs=[pl.BlockSpec((1, c, ht, w), lambda b, i: (b, 0, i, 0))],
        out_specs=pl.BlockSpec((1, ht, w), lambda b, i: (b, i, 0)),
        out_shape=jax.ShapeDtypeStruct((n, h, w), jnp.float32),
    )(x)

    sc_body = functools.partial(
        _sc_select_body, n=n, h=h, w=w, hout=hout, wout=wout,
        kkeep=kkeep, nrounds=nrounds)
    kth_flat = pl.kernel(
        sc_body,
        out_type=jax.ShapeDtypeStruct((nprob * 16,), jnp.int32),
        mesh=plsc.VectorSubcoreMesh(core_axis_name="c", subcore_axis_name="s"),
        compiler_params=pltpu.CompilerParams(needs_layout_passes=False),
        scratch_types=[
            pltpu.VMEM((_CH, w), jnp.float32),
            pltpu.VMEM((65536,), jnp.int32),
            pltpu.VMEM((4096,), jnp.int32),
            pltpu.VMEM((16,), jnp.int32),
        ],
    )(tmean.reshape(n * h, w))
    kth = kth_flat.reshape(nprob, 16)[:, 0].reshape(n, _KH * _KW)
    kth = jnp.broadcast_to(
        jnp.pad(kth, ((0, 0), (0, 128 - _KH * _KW)))[:, None, :], (n, 8, 128))

    tau_arr = jnp.full((8, 128), tau, dtype=jnp.float32)
    out = pl.pallas_call(
        functools.partial(_apply_body, hout=hout, wout=wout, ht=ht),
        grid=(n, nh),
        in_specs=[
            pl.BlockSpec((1, c, ht, w), lambda b, i: (b, 0, i, 0)),
            pl.BlockSpec((1, ht, w), lambda b, i: (b, i, 0)),
            pl.BlockSpec((1, 8, 128), lambda b, i: (b, 0, 0)),
            pl.BlockSpec((8, 128), lambda b, i: (0, 0)),
        ],
        out_specs=pl.BlockSpec((1, c, ht, w), lambda b, i: (b, 0, i, 0)),
        out_shape=jax.ShapeDtypeStruct((n, c, h, w), jnp.float32),
    )(x, tmean, kth, tau_arr)
    return out


# speedup vs baseline: 1.1398x; 1.1398x over previous
import functools

import jax
import jax.numpy as jnp
from jax import lax
from jax.experimental import pallas as pl
from jax.experimental.pallas import tpu as pltpu
from jax.experimental.pallas import tpu_sc as plsc

_TOPK = 0.1
_KH, _KW = 5, 5
_INT_MAX = 2147483647
_SIGN = -2147483648
_NC, _NS = 2, 16
_NW = _NC * _NS
_CH = 48


def _sortable_key(f):
    b = jax.lax.bitcast_convert_type(f, jnp.int32)
    return jnp.where(b < 0, b ^ 0x7FFFFFFF, b)


def _mean_body(x_ref, t_ref):
    t_ref[0] = _sortable_key(jnp.mean(x_ref[0], axis=0))


def _sc_select_body(t_hbm, kth_hbm, rowbuf, hfine, hcoarse, colv, outv, *,
                    n, h, w, hout, wout, kkeep, nrounds):
    nprob = n * _KH * _KW
    wid = lax.axis_index("s") * _NC + lax.axis_index("c")
    lane = lax.broadcasted_iota(jnp.int32, (16,), 0)
    zeros16 = jnp.zeros((16,), jnp.int32)
    nvec = w // 16
    unroll = 4 if nvec % 4 == 0 else 1

    def walk(hist_ref, base, ngroups, krank):
        def step(t, carry):
            tot, bstar, kin = carry
            binv = base + (ngroups - 1 - t) * 16 + lane
            acc = zeros16
            for l2 in range(16):
                acc = acc + plsc.load_gather(hist_ref, [binv * 16 + l2])
            desc = lax.rev(jnp.cumsum(lax.rev(acc, (0,))), (0,))
            above_incl = tot + desc
            above_excl = above_incl - acc
            found = jnp.logical_and(above_incl >= krank, above_excl < krank)
            bstar = bstar + jnp.sum(jnp.where(found, binv + 1, 0))
            kin = kin + jnp.sum(jnp.where(found, krank - above_excl, 0))
            return tot + jnp.sum(acc), bstar, kin
        _, bstar, kin = lax.fori_loop(
            0, ngroups, step, (jnp.int32(0), jnp.int32(-1), jnp.int32(0)))
        return bstar, kin

    def scan_pass(b, wi, wj, rank, shift, pfx_shift, pfx):
        def clr_f(i, c):
            for k in range(8):
                plsc.store_scatter(hfine, [i * 128 + k * 16 + lane], zeros16)
            return c
        lax.fori_loop(0, 512, clr_f, 0)

        def clr_c(i, c):
            for k in range(8):
                plsc.store_scatter(hcoarse, [i * 128 + k * 16 + lane],
                                   zeros16)
            return c
        lax.fori_loop(0, 32, clr_c, 0)

        def vec_work(rbase, cb):
            bits = plsc.load_gather(rowbuf, [rbase + cb + lane])
            cm = plsc.load_gather(colv, [cb + lane])
            ukey = bits ^ _SIGN
            if shift == 20:
                sl = lax.shift_right_logical(ukey, 20)
            elif shift == 8:
                sl = jnp.bitwise_and(lax.shift_right_logical(ukey, 8), 0xFFF)
            else:
                sl = jnp.left_shift(jnp.bitwise_and(ukey, 0xFF), 4)
            if pfx is None:
                val = cm
            else:
                val = jnp.where(
                    lax.shift_right_logical(ukey, pfx_shift) == pfx, cm, 0)
            faddr = jnp.left_shift(sl, 4) + lane
            caddr = jnp.bitwise_and(sl, ~0xF) + lane
            plsc.addupdate_scatter(hcoarse, [caddr], val)
            plsc.addupdate_scatter(hfine, [faddr], val)

        def chunk_body(ck, c):
            pltpu.sync_copy(
                t_hbm.at[pl.ds((b * h + ck * _CH) * w, _CH * w)], rowbuf)

            def row_body(r, c2):
                grow = ck * _CH + r

                @pl.when(jnp.logical_and(grow >= wi, grow < wi + hout))
                def _():
                    rbase = r * w

                    def vec_body(v, c3):
                        for k in range(unroll):
                            vec_work(rbase, (v * unroll + k) * 16)
                        return c3
                    lax.fori_loop(0, nvec // unroll, vec_body, 0)
                return c2
            lax.fori_loop(0, _CH, row_body, 0)
            return c
        lax.fori_loop(0, h // _CH, chunk_body, 0)

        cstar, kin = walk(hcoarse, jnp.int32(0), 16, rank)
        fstar, kin2 = walk(hfine, cstar * 16, 1, kin)
        return fstar, kin2

    def round_body(rnd, carry):
        p = rnd * _NW + wid

        @pl.when(p < nprob)
        def _():
            b = p // (_KH * _KW)
            wwin = p % (_KH * _KW)
            wi = wwin // _KW
            wj = wwin % _KW

            def colv_init(v, c):
                col = v * 16 + lane
                cm = jnp.logical_and(col >= wj, col < wj + wout)
                plsc.store_scatter(colv, [col], cm.astype(jnp.int32))
                return c
            lax.fori_loop(0, nvec, colv_init, 0)

            f1, k2 = scan_pass(b, wi, wj, jnp.int32(kkeep), 20, 0, None)
            f2, k3 = scan_pass(b, wi, wj, k2, 8, 20, f1)
            pfx2 = jnp.bitwise_or(jnp.left_shift(f1, 12), f2)
            f3, _ = scan_pass(b, wi, wj, k3, 0, 8, pfx2)
            ukey = jnp.bitwise_or(jnp.left_shift(pfx2, 8),
                                  lax.shift_right_logical(f3, 4))
            skey = ukey ^ _SIGN
            outv[...] = jnp.full((16,), skey, jnp.int32)
            pltpu.sync_copy(outv, kth_hbm.at[pl.ds(p * 16, 16)])
        return carry

    lax.fori_loop(0, nrounds, round_body, 0)


def _apply_body(x_ref, t_ref, k_ref, tau_ref, o_ref, *, hout, wout, ht):
    i0 = pl.program_id(1)
    key = t_ref[0]
    H, W = key.shape

    p = i0 * ht + jax.lax.broadcasted_iota(jnp.int32, (H, 1), 0)
    q = jax.lax.broadcasted_iota(jnp.int32, (1, W), 1)
    big = jnp.int32(_INT_MAX)
    thr = None
    for j in range(_KW):
        cm = jnp.logical_and(q >= j, q < j + wout)
        rmin = None
        for i in range(_KH):
            rm = jnp.logical_and(p >= i, p < i + hout)
            v = jnp.where(rm, k_ref[0, 0, i * _KW + j], big)
            rmin = v if rmin is None else jnp.minimum(rmin, v)
        v = jnp.where(cm, rmin, big)
        thr = v if thr is None else jnp.minimum(thr, v)

    mask = (key >= thr).astype(jnp.float32)
    tau = tau_ref[0, 0]
    wmap = mask * tau + (1.0 - tau)
    o_ref[0] = x_ref[0] * wmap[None]


@jax.jit
def kernel(x, tau):
    n, c, h, w = x.shape
    hout, wout = h - _KH + 1, w - _KW + 1
    kkeep = max(int(_TOPK * (hout * wout)), 1)
    nprob = n * _KH * _KW
    nrounds = (nprob + _NW - 1) // _NW

    ht = 32 if h % 32 == 0 else h
    nh = h // ht

    tkey = pl.pallas_call(
        _mean_body,
        grid=(n, nh),
        in_specs=[pl.BlockSpec((1, c, ht, w), lambda b, i: (b, 0, i, 0))],
        out_specs=pl.BlockSpec((1, ht, w), lambda b, i: (b, i, 0)),
        out_shape=jax.ShapeDtypeStruct((n, h, w), jnp.int32),
    )(x)

    sc_body = functools.partial(
        _sc_select_body, n=n, h=h, w=w, hout=hout, wout=wout,
        kkeep=kkeep, nrounds=nrounds)
    kth_flat = pl.kernel(
        sc_body,
        out_type=jax.ShapeDtypeStruct((nprob * 16,), jnp.int32),
        mesh=plsc.VectorSubcoreMesh(core_axis_name="c", subcore_axis_name="s"),
        compiler_params=pltpu.CompilerParams(needs_layout_passes=False),
        scratch_types=[
            pltpu.VMEM((_CH * w,), jnp.int32),
            pltpu.VMEM((65536,), jnp.int32),
            pltpu.VMEM((4096,), jnp.int32),
            pltpu.VMEM((w,), jnp.int32),
            pltpu.VMEM((16,), jnp.int32),
        ],
    )(tkey.reshape(n * h * w))
    kth = kth_flat.reshape(nprob, 16)[:, 0].reshape(n, _KH * _KW)
    kth = jnp.broadcast_to(
        jnp.pad(kth, ((0, 0), (0, 128 - _KH * _KW)))[:, None, :], (n, 8, 128))

    tau_arr = jnp.full((8, 128), tau, dtype=jnp.float32)
    out = pl.pallas_call(
        functools.partial(_apply_body, hout=hout, wout=wout, ht=ht),
        grid=(n, nh),
        in_specs=[
            pl.BlockSpec((1, c, ht, w), lambda b, i: (b, 0, i, 0)),
            pl.BlockSpec((1, ht, w), lambda b, i: (b, i, 0)),
            pl.BlockSpec((1, 8, 128), lambda b, i: (b, 0, 0)),
            pl.BlockSpec((8, 128), lambda b, i: (0, 0)),
        ],
        out_specs=pl.BlockSpec((1, c, ht, w), lambda b, i: (b, 0, i, 0)),
        out_shape=jax.ShapeDtypeStruct((n, c, h, w), jnp.float32),
    )(x, tkey, kth, tau_arr)
    return out


# speedup vs baseline: 1.2282x; 1.0775x over previous
import functools

import jax
import jax.numpy as jnp
from jax import lax
from jax.experimental import pallas as pl
from jax.experimental.pallas import tpu as pltpu
from jax.experimental.pallas import tpu_sc as plsc

_TOPK = 0.1
_KH, _KW = 5, 5
_INT_MAX = 2147483647
_SIGN = -2147483648
_NC, _NS = 2, 16
_NW = _NC * _NS
_CH = 48


def _sortable_key(f):
    b = jax.lax.bitcast_convert_type(f, jnp.int32)
    return jnp.where(b < 0, b ^ 0x7FFFFFFF, b)


def _mean_body(x_ref, t_ref):
    t_ref[0] = _sortable_key(jnp.mean(x_ref[0], axis=0))


def _sc_select_body(t_hbm, kth_hbm, rowbuf, rowbufb, hfine, hcoarse, colv,
                    outv, sema, semb, *, n, h, w, hout, wout, kkeep, nrounds):
    nprob = n * _KH * _KW
    wid = lax.axis_index("s") * _NC + lax.axis_index("c")
    lane = lax.broadcasted_iota(jnp.int32, (16,), 0)
    zeros16 = jnp.zeros((16,), jnp.int32)
    nvec = w // 16
    unroll = 4 if nvec % 4 == 0 else 1

    def walk(hist_ref, base, ngroups, krank):
        def step(t, carry):
            tot, bstar, kin = carry
            binv = base + (ngroups - 1 - t) * 16 + lane
            acc = zeros16
            for l2 in range(16):
                acc = acc + plsc.load_gather(hist_ref, [binv * 16 + l2])
            desc = lax.rev(jnp.cumsum(lax.rev(acc, (0,))), (0,))
            above_incl = tot + desc
            above_excl = above_incl - acc
            found = jnp.logical_and(above_incl >= krank, above_excl < krank)
            bstar = bstar + jnp.sum(jnp.where(found, binv + 1, 0))
            kin = kin + jnp.sum(jnp.where(found, krank - above_excl, 0))
            return tot + jnp.sum(acc), bstar, kin
        _, bstar, kin = lax.fori_loop(
            0, ngroups, step, (jnp.int32(0), jnp.int32(-1), jnp.int32(0)))
        return bstar, kin

    def scan_pass(b, wi, wj, rank, shift, pfx_shift, pfx):
        def clr_f(i, c):
            for k in range(8):
                plsc.store_scatter(hfine, [i * 128 + k * 16 + lane], zeros16)
            return c
        lax.fori_loop(0, 512, clr_f, 0)

        def clr_c(i, c):
            for k in range(8):
                plsc.store_scatter(hcoarse, [i * 128 + k * 16 + lane],
                                   zeros16)
            return c
        lax.fori_loop(0, 32, clr_c, 0)

        def vec_work(buf, rbase, cb):
            bits = plsc.load_gather(buf, [rbase + cb + lane])
            cm = plsc.load_gather(colv, [cb + lane])
            ukey = bits ^ _SIGN
            if shift == 20:
                sl = lax.shift_right_logical(ukey, 20)
            elif shift == 8:
                sl = jnp.bitwise_and(lax.shift_right_logical(ukey, 8), 0xFFF)
            else:
                sl = jnp.left_shift(jnp.bitwise_and(ukey, 0xFF), 4)
            if pfx is None:
                val = cm
            else:
                val = jnp.where(
                    lax.shift_right_logical(ukey, pfx_shift) == pfx, cm, 0)
            faddr = jnp.left_shift(sl, 4) + lane
            caddr = jnp.bitwise_and(sl, ~0xF) + lane
            plsc.addupdate_scatter(hcoarse, [caddr], val)
            plsc.addupdate_scatter(hfine, [faddr], val)

        def chunk_src(ck):
            return t_hbm.at[pl.ds((b * h + ck * _CH) * w, _CH * w)]

        def process(buf, ck):
            def row_body(r, c2):
                grow = ck * _CH + r

                @pl.when(jnp.logical_and(grow >= wi, grow < wi + hout))
                def _():
                    rbase = r * w

                    def vec_body(v, c3):
                        for k in range(unroll):
                            vec_work(buf, rbase, (v * unroll + k) * 16)
                        return c3
                    lax.fori_loop(0, nvec // unroll, vec_body, 0)
                return c2
            lax.fori_loop(0, _CH, row_body, 0)

        nck = h // _CH
        pltpu.async_copy(chunk_src(0), rowbuf, sema)

        def group_body(g, c):
            pltpu.async_copy(chunk_src(2 * g + 1), rowbufb, semb)
            pltpu.make_async_copy(chunk_src(2 * g), rowbuf, sema).wait()
            process(rowbuf, 2 * g)

            @pl.when(g + 1 < nck // 2)
            def _():
                pltpu.async_copy(chunk_src(2 * g + 2), rowbuf, sema)
            pltpu.make_async_copy(chunk_src(2 * g + 1), rowbufb, semb).wait()
            process(rowbufb, 2 * g + 1)
            return c
        lax.fori_loop(0, nck // 2, group_body, 0)

        cstar, kin = walk(hcoarse, jnp.int32(0), 16, rank)
        fstar, kin2 = walk(hfine, cstar * 16, 1, kin)
        return fstar, kin2

    def round_body(rnd, carry):
        p = rnd * _NW + wid

        @pl.when(p < nprob)
        def _():
            b = p // (_KH * _KW)
            wwin = p % (_KH * _KW)
            wi = wwin // _KW
            wj = wwin % _KW

            def colv_init(v, c):
                col = v * 16 + lane
                cm = jnp.logical_and(col >= wj, col < wj + wout)
                plsc.store_scatter(colv, [col], cm.astype(jnp.int32))
                return c
            lax.fori_loop(0, nvec, colv_init, 0)

            f1, k2 = scan_pass(b, wi, wj, jnp.int32(kkeep), 20, 0, None)
            f2, k3 = scan_pass(b, wi, wj, k2, 8, 20, f1)
            pfx2 = jnp.bitwise_or(jnp.left_shift(f1, 12), f2)
            f3, _ = scan_pass(b, wi, wj, k3, 0, 8, pfx2)
            ukey = jnp.bitwise_or(jnp.left_shift(pfx2, 8),
                                  lax.shift_right_logical(f3, 4))
            skey = ukey ^ _SIGN
            outv[...] = jnp.full((16,), skey, jnp.int32)
            pltpu.sync_copy(outv, kth_hbm.at[pl.ds(p * 16, 16)])
        return carry

    lax.fori_loop(0, nrounds, round_body, 0)


def _apply_body(x_ref, t_ref, k_ref, tau_ref, o_ref, *, hout, wout, ht):
    i0 = pl.program_id(1)
    key = t_ref[0]
    H, W = key.shape

    p = i0 * ht + jax.lax.broadcasted_iota(jnp.int32, (H, 1), 0)
    q = jax.lax.broadcasted_iota(jnp.int32, (1, W), 1)
    big = jnp.int32(_INT_MAX)
    thr = None
    for j in range(_KW):
        cm = jnp.logical_and(q >= j, q < j + wout)
        rmin = None
        for i in range(_KH):
            rm = jnp.logical_and(p >= i, p < i + hout)
            v = jnp.where(rm, k_ref[0, 0, i * _KW + j], big)
            rmin = v if rmin is None else jnp.minimum(rmin, v)
        v = jnp.where(cm, rmin, big)
        thr = v if thr is None else jnp.minimum(thr, v)

    mask = (key >= thr).astype(jnp.float32)
    tau = tau_ref[0, 0]
    wmap = mask * tau + (1.0 - tau)
    o_ref[0] = x_ref[0] * wmap[None]


@jax.jit
def kernel(x, tau):
    n, c, h, w = x.shape
    hout, wout = h - _KH + 1, w - _KW + 1
    kkeep = max(int(_TOPK * (hout * wout)), 1)
    nprob = n * _KH * _KW
    nrounds = (nprob + _NW - 1) // _NW

    ht = 32 if h % 32 == 0 else h
    nh = h // ht

    tkey = pl.pallas_call(
        _mean_body,
        grid=(n, nh),
        in_specs=[pl.BlockSpec((1, c, ht, w), lambda b, i: (b, 0, i, 0))],
        out_specs=pl.BlockSpec((1, ht, w), lambda b, i: (b, i, 0)),
        out_shape=jax.ShapeDtypeStruct((n, h, w), jnp.int32),
    )(x)

    sc_body = functools.partial(
        _sc_select_body, n=n, h=h, w=w, hout=hout, wout=wout,
        kkeep=kkeep, nrounds=nrounds)
    kth_flat = pl.kernel(
        sc_body,
        out_type=jax.ShapeDtypeStruct((nprob * 16,), jnp.int32),
        mesh=plsc.VectorSubcoreMesh(core_axis_name="c", subcore_axis_name="s"),
        compiler_params=pltpu.CompilerParams(needs_layout_passes=False),
        scratch_types=[
            pltpu.VMEM((_CH * w,), jnp.int32),
            pltpu.VMEM((_CH * w,), jnp.int32),
            pltpu.VMEM((65536,), jnp.int32),
            pltpu.VMEM((4096,), jnp.int32),
            pltpu.VMEM((w,), jnp.int32),
            pltpu.VMEM((16,), jnp.int32),
            pltpu.SemaphoreType.DMA,
            pltpu.SemaphoreType.DMA,
        ],
    )(tkey.reshape(n * h * w))
    kth = kth_flat.reshape(nprob, 16)[:, 0].reshape(n, _KH * _KW)
    kth = jnp.broadcast_to(
        jnp.pad(kth, ((0, 0), (0, 128 - _KH * _KW)))[:, None, :], (n, 8, 128))

    tau_arr = jnp.full((8, 128), tau, dtype=jnp.float32)
    out = pl.pallas_call(
        functools.partial(_apply_body, hout=hout, wout=wout, ht=ht),
        grid=(n, nh),
        in_specs=[
            pl.BlockSpec((1, c, ht, w), lambda b, i: (b, 0, i, 0)),
            pl.BlockSpec((1, ht, w), lambda b, i: (b, i, 0)),
            pl.BlockSpec((1, 8, 128), lambda b, i: (b, 0, 0)),
            pl.BlockSpec((8, 128), lambda b, i: (0, 0)),
        ],
        out_specs=pl.BlockSpec((1, c, ht, w), lambda b, i: (b, 0, i, 0)),
        out_shape=jax.ShapeDtypeStruct((n, c, h, w), jnp.float32),
    )(x, tkey, kth, tau_arr)
    return out


# speedup vs baseline: 1.2441x; 1.0129x over previous
import functools

import jax
import jax.numpy as jnp
from jax import lax
from jax.experimental import pallas as pl
from jax.experimental.pallas import tpu as pltpu
from jax.experimental.pallas import tpu_sc as plsc

_TOPK = 0.1
_KH, _KW = 5, 5
_INT_MAX = 2147483647
_SIGN = -2147483648
_NC, _NS = 2, 16
_NW = _NC * _NS
_CH = 64


def _sortable_key(f):
    b = jax.lax.bitcast_convert_type(f, jnp.int32)
    return jnp.where(b < 0, b ^ 0x7FFFFFFF, b)


def _mean_body(x_ref, t_ref):
    t_ref[0] = _sortable_key(jnp.mean(x_ref[0], axis=0))


def _sc_select_body(t_hbm, kth_hbm, rowbuf, rowbufb, hfine, hcoarse, colv,
                    outv, sema, semb, *, n, h, w, hout, wout, kkeep, nrounds):
    nprob = n * _KH * _KW
    wid = lax.axis_index("s") * _NC + lax.axis_index("c")
    lane = lax.broadcasted_iota(jnp.int32, (16,), 0)
    zeros16 = jnp.zeros((16,), jnp.int32)
    nvec = w // 16
    unroll = 8 if nvec % 8 == 0 else (4 if nvec % 4 == 0 else 1)

    def walk(hist_ref, base, ngroups, krank):
        def step(t, carry):
            tot, bstar, kin = carry
            binv = base + (ngroups - 1 - t) * 16 + lane
            acc = zeros16
            for l2 in range(16):
                acc = acc + plsc.load_gather(hist_ref, [binv * 16 + l2])
            desc = lax.rev(jnp.cumsum(lax.rev(acc, (0,))), (0,))
            above_incl = tot + desc
            above_excl = above_incl - acc
            found = jnp.logical_and(above_incl >= krank, above_excl < krank)
            bstar = bstar + jnp.sum(jnp.where(found, binv + 1, 0))
            kin = kin + jnp.sum(jnp.where(found, krank - above_excl, 0))
            return tot + jnp.sum(acc), bstar, kin
        _, bstar, kin = lax.fori_loop(
            0, ngroups, step, (jnp.int32(0), jnp.int32(-1), jnp.int32(0)))
        return bstar, kin

    def scan_pass(b, wi, wj, rank, shift, pfx_shift, pfx):
        def clr_f(i, c):
            for k in range(8):
                plsc.store_scatter(hfine, [i * 128 + k * 16 + lane], zeros16)
            return c
        lax.fori_loop(0, 512, clr_f, 0)

        def clr_c(i, c):
            for k in range(8):
                plsc.store_scatter(hcoarse, [i * 128 + k * 16 + lane],
                                   zeros16)
            return c
        lax.fori_loop(0, 32, clr_c, 0)

        def vec_work(buf, rbase, cb):
            bits = plsc.load_gather(buf, [rbase + cb + lane])
            cm = plsc.load_gather(colv, [cb + lane])
            ukey = bits ^ _SIGN
            if shift == 20:
                sl = lax.shift_right_logical(ukey, 20)
            elif shift == 8:
                sl = jnp.bitwise_and(lax.shift_right_logical(ukey, 8), 0xFFF)
            else:
                sl = jnp.left_shift(jnp.bitwise_and(ukey, 0xFF), 4)
            if pfx is None:
                val = cm
            else:
                val = jnp.where(
                    lax.shift_right_logical(ukey, pfx_shift) == pfx, cm, 0)
            faddr = jnp.left_shift(sl, 4) + lane
            caddr = jnp.bitwise_and(sl, ~0xF) + lane
            plsc.addupdate_scatter(hcoarse, [caddr], val)
            plsc.addupdate_scatter(hfine, [faddr], val)

        def chunk_src(ck):
            return t_hbm.at[pl.ds((b * h + ck * _CH) * w, _CH * w)]

        def process(buf, ck):
            def row_body(r, c2):
                grow = ck * _CH + r

                @pl.when(jnp.logical_and(grow >= wi, grow < wi + hout))
                def _():
                    rbase = r * w

                    def vec_body(v, c3):
                        for k in range(unroll):
                            vec_work(buf, rbase, (v * unroll + k) * 16)
                        return c3
                    lax.fori_loop(0, nvec // unroll, vec_body, 0)
                return c2
            lax.fori_loop(0, _CH, row_body, 0)

        nck = h // _CH
        pltpu.async_copy(chunk_src(0), rowbuf, sema)

        def group_body(g, c):
            pltpu.async_copy(chunk_src(2 * g + 1), rowbufb, semb)
            pltpu.make_async_copy(chunk_src(2 * g), rowbuf, sema).wait()
            process(rowbuf, 2 * g)

            @pl.when(g + 1 < nck // 2)
            def _():
                pltpu.async_copy(chunk_src(2 * g + 2), rowbuf, sema)
            pltpu.make_async_copy(chunk_src(2 * g + 1), rowbufb, semb).wait()
            process(rowbufb, 2 * g + 1)
            return c
        lax.fori_loop(0, nck // 2, group_body, 0)

        cstar, kin = walk(hcoarse, jnp.int32(0), 16, rank)
        fstar, kin2 = walk(hfine, cstar * 16, 1, kin)
        return fstar, kin2

    def round_body(rnd, carry):
        p = rnd * _NW + wid

        @pl.when(p < nprob)
        def _():
            b = p // (_KH * _KW)
            wwin = p % (_KH * _KW)
            wi = wwin // _KW
            wj = wwin % _KW

            def colv_init(v, c):
                col = v * 16 + lane
                cm = jnp.logical_and(col >= wj, col < wj + wout)
                plsc.store_scatter(colv, [col], cm.astype(jnp.int32))
                return c
            lax.fori_loop(0, nvec, colv_init, 0)

            f1, k2 = scan_pass(b, wi, wj, jnp.int32(kkeep), 20, 0, None)
            f2, k3 = scan_pass(b, wi, wj, k2, 8, 20, f1)
            pfx2 = jnp.bitwise_or(jnp.left_shift(f1, 12), f2)
            f3, _ = scan_pass(b, wi, wj, k3, 0, 8, pfx2)
            ukey = jnp.bitwise_or(jnp.left_shift(pfx2, 8),
                                  lax.shift_right_logical(f3, 4))
            skey = ukey ^ _SIGN
            outv[...] = jnp.full((16,), skey, jnp.int32)
            pltpu.sync_copy(outv, kth_hbm.at[pl.ds(p * 16, 16)])
        return carry

    lax.fori_loop(0, nrounds, round_body, 0)


def _apply_body(x_ref, t_ref, k_ref, tau_ref, o_ref, *, hout, wout, ht):
    i0 = pl.program_id(1)
    key = t_ref[0]
    H, W = key.shape

    p = i0 * ht + jax.lax.broadcasted_iota(jnp.int32, (H, 1), 0)
    q = jax.lax.broadcasted_iota(jnp.int32, (1, W), 1)
    big = jnp.int32(_INT_MAX)
    thr = None
    for j in range(_KW):
        cm = jnp.logical_and(q >= j, q < j + wout)
        rmin = None
        for i in range(_KH):
            rm = jnp.logical_and(p >= i, p < i + hout)
            v = jnp.where(rm, k_ref[0, 0, i * _KW + j], big)
            rmin = v if rmin is None else jnp.minimum(rmin, v)
        v = jnp.where(cm, rmin, big)
        thr = v if thr is None else jnp.minimum(thr, v)

    mask = (key >= thr).astype(jnp.float32)
    tau = tau_ref[0, 0]
    wmap = mask * tau + (1.0 - tau)
    o_ref[0] = x_ref[0] * wmap[None]


@jax.jit
def kernel(x, tau):
    n, c, h, w = x.shape
    hout, wout = h - _KH + 1, w - _KW + 1
    kkeep = max(int(_TOPK * (hout * wout)), 1)
    nprob = n * _KH * _KW
    nrounds = (nprob + _NW - 1) // _NW

    ht = 32 if h % 32 == 0 else h
    nh = h // ht

    tkey = pl.pallas_call(
        _mean_body,
        grid=(n, nh),
        in_specs=[pl.BlockSpec((1, c, ht, w), lambda b, i: (b, 0, i, 0))],
        out_specs=pl.BlockSpec((1, ht, w), lambda b, i: (b, i, 0)),
        out_shape=jax.ShapeDtypeStruct((n, h, w), jnp.int32),
    )(x)

    sc_body = functools.partial(
        _sc_select_body, n=n, h=h, w=w, hout=hout, wout=wout,
        kkeep=kkeep, nrounds=nrounds)
    kth_flat = pl.kernel(
        sc_body,
        out_type=jax.ShapeDtypeStruct((nprob * 16,), jnp.int32),
        mesh=plsc.VectorSubcoreMesh(core_axis_name="c", subcore_axis_name="s"),
        compiler_params=pltpu.CompilerParams(needs_layout_passes=False),
        scratch_types=[
            pltpu.VMEM((_CH * w,), jnp.int32),
            pltpu.VMEM((_CH * w,), jnp.int32),
            pltpu.VMEM((65536,), jnp.int32),
            pltpu.VMEM((4096,), jnp.int32),
            pltpu.VMEM((w,), jnp.int32),
            pltpu.VMEM((16,), jnp.int32),
            pltpu.SemaphoreType.DMA,
            pltpu.SemaphoreType.DMA,
        ],
    )(tkey.reshape(n * h * w))
    kth = kth_flat.reshape(nprob, 16)[:, 0].reshape(n, _KH * _KW)
    kth = jnp.broadcast_to(
        jnp.pad(kth, ((0, 0), (0, 128 - _KH * _KW)))[:, None, :], (n, 8, 128))

    tau_arr = jnp.full((8, 128), tau, dtype=jnp.float32)
    out = pl.pallas_call(
        functools.partial(_apply_body, hout=hout, wout=wout, ht=ht),
        grid=(n, nh),
        in_specs=[
            pl.BlockSpec((1, c, ht, w), lambda b, i: (b, 0, i, 0)),
            pl.BlockSpec((1, ht, w), lambda b, i: (b, i, 0)),
            pl.BlockSpec((1, 8, 128), lambda b, i: (b, 0, 0)),
            pl.BlockSpec((8, 128), lambda b, i: (0, 0)),
        ],
        out_specs=pl.BlockSpec((1, c, ht, w), lambda b, i: (b, 0, i, 0)),
        out_shape=jax.ShapeDtypeStruct((n, c, h, w), jnp.float32),
    )(x, tkey, kth, tau_arr)
    return out


# speedup vs baseline: 1.2882x; 1.0355x over previous
import functools

import jax
import jax.numpy as jnp
from jax import lax
from jax.experimental import pallas as pl
from jax.experimental.pallas import tpu as pltpu
from jax.experimental.pallas import tpu_sc as plsc

_TOPK = 0.1
_KH, _KW = 5, 5
_INT_MAX = 2147483647
_SIGN = -2147483648
_NC, _NS = 2, 16
_NW = _NC * _NS
_CH = 64


def _sortable_key(f):
    b = jax.lax.bitcast_convert_type(f, jnp.int32)
    return jnp.where(b < 0, b ^ 0x7FFFFFFF, b)


def _mean_body(x_ref, t_ref):
    t_ref[0] = _sortable_key(jnp.mean(x_ref[0], axis=0))


def _sc_select_body(t_hbm, kth_hbm, rowbuf, rowbufb, hfine, colv,
                    outv, sema, semb, *, n, h, w, hout, wout, kkeep, nrounds):
    nprob = n * _KH * _KW
    wid = lax.axis_index("s") * _NC + lax.axis_index("c")
    lane = lax.broadcasted_iota(jnp.int32, (16,), 0)
    zeros16 = jnp.zeros((16,), jnp.int32)
    nvec = w // 16
    unroll = 8 if nvec % 8 == 0 else (4 if nvec % 4 == 0 else 1)

    def walk(hist_ref, base, ngroups, krank):
        def step(t, carry):
            tot, bstar, kin = carry
            binv = base + (ngroups - 1 - t) * 16 + lane
            acc = zeros16
            for l2 in range(16):
                acc = acc + plsc.load_gather(hist_ref, [binv * 16 + l2])
            desc = lax.rev(jnp.cumsum(lax.rev(acc, (0,))), (0,))
            above_incl = tot + desc
            above_excl = above_incl - acc
            found = jnp.logical_and(above_incl >= krank, above_excl < krank)
            bstar = bstar + jnp.sum(jnp.where(found, binv + 1, 0))
            kin = kin + jnp.sum(jnp.where(found, krank - above_excl, 0))
            return tot + jnp.sum(acc), bstar, kin
        _, bstar, kin = lax.fori_loop(
            0, ngroups, step, (jnp.int32(0), jnp.int32(-1), jnp.int32(0)))
        return bstar, kin

    def scan_pass(b, wi, wj, rank, shift, pfx_shift, pfx):
        def clr_f(i, c):
            for k in range(8):
                plsc.store_scatter(hfine, [i * 128 + k * 16 + lane], zeros16)
            return c
        lax.fori_loop(0, 512, clr_f, 0)

        def vec_work(buf, rbase, cb):
            bits = plsc.load_gather(buf, [rbase + cb + lane])
            cm = plsc.load_gather(colv, [cb + lane])
            ukey = bits ^ _SIGN
            if shift == 20:
                sl = lax.shift_right_logical(ukey, 20)
            elif shift == 8:
                sl = jnp.bitwise_and(lax.shift_right_logical(ukey, 8), 0xFFF)
            else:
                sl = jnp.left_shift(jnp.bitwise_and(ukey, 0xFF), 4)
            if pfx is None:
                val = cm
            else:
                val = jnp.where(
                    lax.shift_right_logical(ukey, pfx_shift) == pfx, cm, 0)
            plsc.addupdate_scatter(hfine, [jnp.left_shift(sl, 4) + lane],
                                   val)

        def chunk_src(ck):
            return t_hbm.at[pl.ds((b * h + ck * _CH) * w, _CH * w)]

        def process(buf, ck):
            def row_body(r, c2):
                grow = ck * _CH + r

                @pl.when(jnp.logical_and(grow >= wi, grow < wi + hout))
                def _():
                    rbase = r * w

                    def vec_body(v, c3):
                        for k in range(unroll):
                            vec_work(buf, rbase, (v * unroll + k) * 16)
                        return c3
                    lax.fori_loop(0, nvec // unroll, vec_body, 0)
                return c2
            lax.fori_loop(0, _CH, row_body, 0)

        nck = h // _CH
        pltpu.async_copy(chunk_src(0), rowbuf, sema)

        def group_body(g, c):
            pltpu.async_copy(chunk_src(2 * g + 1), rowbufb, semb)
            pltpu.make_async_copy(chunk_src(2 * g), rowbuf, sema).wait()
            process(rowbuf, 2 * g)

            @pl.when(g + 1 < nck // 2)
            def _():
                pltpu.async_copy(chunk_src(2 * g + 2), rowbuf, sema)
            pltpu.make_async_copy(chunk_src(2 * g + 1), rowbufb, semb).wait()
            process(rowbufb, 2 * g + 1)
            return c
        lax.fori_loop(0, nck // 2, group_body, 0)

        return walk(hfine, jnp.int32(0), 256, rank)

    def round_body(rnd, carry):
        p = rnd * _NW + wid

        @pl.when(p < nprob)
        def _():
            b = p // (_KH * _KW)
            wwin = p % (_KH * _KW)
            wi = wwin // _KW
            wj = wwin % _KW

            def colv_init(v, c):
                col = v * 16 + lane
                cm = jnp.logical_and(col >= wj, col < wj + wout)
                plsc.store_scatter(colv, [col], cm.astype(jnp.int32))
                return c
            lax.fori_loop(0, nvec, colv_init, 0)

            f1, k2 = scan_pass(b, wi, wj, jnp.int32(kkeep), 20, 0, None)
            f2, k3 = scan_pass(b, wi, wj, k2, 8, 20, f1)
            pfx2 = jnp.bitwise_or(jnp.left_shift(f1, 12), f2)
            f3, _ = scan_pass(b, wi, wj, k3, 0, 8, pfx2)
            ukey = jnp.bitwise_or(jnp.left_shift(pfx2, 8),
                                  lax.shift_right_logical(f3, 4))
            skey = ukey ^ _SIGN
            outv[...] = jnp.full((16,), skey, jnp.int32)
            pltpu.sync_copy(outv, kth_hbm.at[pl.ds(p * 16, 16)])
        return carry

    lax.fori_loop(0, nrounds, round_body, 0)


def _apply_body(x_ref, t_ref, k_ref, tau_ref, o_ref, *, hout, wout, ht):
    i0 = pl.program_id(1)
    key = t_ref[0]
    H, W = key.shape

    p = i0 * ht + jax.lax.broadcasted_iota(jnp.int32, (H, 1), 0)
    q = jax.lax.broadcasted_iota(jnp.int32, (1, W), 1)
    big = jnp.int32(_INT_MAX)
    thr = None
    for j in range(_KW):
        cm = jnp.logical_and(q >= j, q < j + wout)
        rmin = None
        for i in range(_KH):
            rm = jnp.logical_and(p >= i, p < i + hout)
            v = jnp.where(rm, k_ref[0, 0, i * _KW + j], big)
            rmin = v if rmin is None else jnp.minimum(rmin, v)
        v = jnp.where(cm, rmin, big)
        thr = v if thr is None else jnp.minimum(thr, v)

    mask = (key >= thr).astype(jnp.float32)
    tau = tau_ref[0, 0]
    wmap = mask * tau + (1.0 - tau)
    o_ref[0] = x_ref[0] * wmap[None]


@jax.jit
def kernel(x, tau):
    n, c, h, w = x.shape
    hout, wout = h - _KH + 1, w - _KW + 1
    kkeep = max(int(_TOPK * (hout * wout)), 1)
    nprob = n * _KH * _KW
    nrounds = (nprob + _NW - 1) // _NW

    ht = 32 if h % 32 == 0 else h
    nh = h // ht

    tkey = pl.pallas_call(
        _mean_body,
        grid=(n, nh),
        in_specs=[pl.BlockSpec((1, c, ht, w), lambda b, i: (b, 0, i, 0))],
        out_specs=pl.BlockSpec((1, ht, w), lambda b, i: (b, i, 0)),
        out_shape=jax.ShapeDtypeStruct((n, h, w), jnp.int32),
    )(x)

    sc_body = functools.partial(
        _sc_select_body, n=n, h=h, w=w, hout=hout, wout=wout,
        kkeep=kkeep, nrounds=nrounds)
    kth_flat = pl.kernel(
        sc_body,
        out_type=jax.ShapeDtypeStruct((nprob * 16,), jnp.int32),
        mesh=plsc.VectorSubcoreMesh(core_axis_name="c", subcore_axis_name="s"),
        compiler_params=pltpu.CompilerParams(needs_layout_passes=False),
        scratch_types=[
            pltpu.VMEM((_CH * w,), jnp.int32),
            pltpu.VMEM((_CH * w,), jnp.int32),
            pltpu.VMEM((65536,), jnp.int32),
            pltpu.VMEM((w,), jnp.int32),
            pltpu.VMEM((16,), jnp.int32),
            pltpu.SemaphoreType.DMA,
            pltpu.SemaphoreType.DMA,
        ],
    )(tkey.reshape(n * h * w))
    kth = kth_flat.reshape(nprob, 16)[:, 0].reshape(n, _KH * _KW)
    kth = jnp.broadcast_to(
        jnp.pad(kth, ((0, 0), (0, 128 - _KH * _KW)))[:, None, :], (n, 8, 128))

    tau_arr = jnp.full((8, 128), tau, dtype=jnp.float32)
    out = pl.pallas_call(
        functools.partial(_apply_body, hout=hout, wout=wout, ht=ht),
        grid=(n, nh),
        in_specs=[
            pl.BlockSpec((1, c, ht, w), lambda b, i: (b, 0, i, 0)),
            pl.BlockSpec((1, ht, w), lambda b, i: (b, i, 0)),
            pl.BlockSpec((1, 8, 128), lambda b, i: (b, 0, 0)),
            pl.BlockSpec((8, 128), lambda b, i: (0, 0)),
        ],
        out_specs=pl.BlockSpec((1, c, ht, w), lambda b, i: (b, 0, i, 0)),
        out_shape=jax.ShapeDtypeStruct((n, c, h, w), jnp.float32),
    )(x, tkey, kth, tau_arr)
    return out
